# Initial kernel scaffold; baseline (speedup 1.0000x reference)
import jax
import jax.numpy as jnp
from jax.experimental import pallas as pl


def kernel(features, edge_index, W0, b0, W1, b1, W2, b2):
    raise NotImplementedError("write your pallas kernel here")


# speedup vs baseline: 2.6335x; 2.6335x over previous
import functools

import jax
import jax.numpy as jnp
from jax import lax
from jax.experimental import pallas as pl
from jax.experimental.pallas import tpu as pltpu
from jax.experimental.pallas import tpu_sc as plsc

N_NODES = 10000
NPAD = 10240
N_EDGES = 320000
EPAD = 327680
D = 128
NW = 32
CPW = 80
CH = 128
ROWS_PER_SUB = NPAD // 16


def _mesh():
    return plsc.VectorSubcoreMesh(core_axis_name="c", subcore_axis_name="s")


CPS = EPAD // 16 // CH


def _deg_body(src_hbm, dst_hbm, ones_hbm, zeros_hbm, out_hbm,
              idx_v, buf_v, acc):
    c = lax.axis_index("c")
    s = lax.axis_index("s")
    pltpu.sync_copy(zeros_hbm, buf_v)
    for t in range(ROWS_PER_SUB // CH):
        pltpu.sync_copy(buf_v, acc.at[pl.ds(s * ROWS_PER_SUB + t * CH, CH)])

    @pl.when(c == 0)
    def _():
        pltpu.sync_copy(src_hbm.at[s], idx_v)

    @pl.when(c == 1)
    def _():
        pltpu.sync_copy(dst_hbm.at[s], idx_v)

    pltpu.sync_copy(ones_hbm, buf_v)
    plsc.subcore_barrier()

    def body(j, carry):
        pltpu.sync_copy(buf_v, acc.at[idx_v.at[j]], add=True)
        return carry

    lax.fori_loop(0, CPS, body, 0)
    plsc.subcore_barrier()
    pltpu.sync_copy(acc.at[pl.ds(s * ROWS_PER_SUB, ROWS_PER_SUB)],
                    out_hbm.at[c, pl.ds(s * ROWS_PER_SUB, ROWS_PER_SUB)])


def _degrees(srcd, dstd, ones128, zeros128):
    k = pl.kernel(
        _deg_body,
        out_type=jax.ShapeDtypeStruct((2, NPAD, D), jnp.float32),
        mesh=_mesh(),
        scratch_types=[
            pltpu.VMEM((CPS, CH), jnp.int32),
            pltpu.VMEM((CH, D), jnp.float32),
            pltpu.VMEM_SHARED((NPAD, D), jnp.float32),
        ],
    )
    return k(srcd, dstd, ones128, zeros128)


def _agg_body(h_hbm, src_hbm, dst_hbm, zeros_hbm, out_hbm,
              src_v, dst_v, rows_v, acc, sem):
    c = lax.axis_index("c")
    s = lax.axis_index("s")
    wid = s * 2 + c
    pltpu.sync_copy(zeros_hbm, rows_v)
    for t in range(ROWS_PER_SUB // CH):
        pltpu.sync_copy(rows_v, acc.at[pl.ds(s * ROWS_PER_SUB + t * CH, CH)])
    pltpu.sync_copy(src_hbm.at[wid], src_v)
    pltpu.sync_copy(dst_hbm.at[wid], dst_v)
    plsc.subcore_barrier()

    def body(j, carry):
        pltpu.async_copy(h_hbm.at[src_v.at[j]], rows_v, sem).wait()
        pltpu.sync_copy(rows_v, acc.at[dst_v.at[j]], add=True)
        return carry

    lax.fori_loop(0, CPW, body, 0)
    plsc.subcore_barrier()
    pltpu.sync_copy(acc.at[pl.ds(s * ROWS_PER_SUB, ROWS_PER_SUB)],
                    out_hbm.at[c, pl.ds(s * ROWS_PER_SUB, ROWS_PER_SUB)])


def _aggregate(h, srcp, dstp, zeros128):
    k = pl.kernel(
        _agg_body,
        out_type=jax.ShapeDtypeStruct((2, NPAD, D), jnp.float32),
        mesh=_mesh(),
        scratch_types=[
            pltpu.VMEM((CPW, CH), jnp.int32),
            pltpu.VMEM((CPW, CH), jnp.int32),
            pltpu.VMEM((CH, D), jnp.float32),
            pltpu.VMEM_SHARED((NPAD, D), jnp.float32),
            pltpu.SemaphoreType.DMA,
        ],
    )
    return k(h, srcp, dstp, zeros128)


BLK = 1024


def _mm_first_body(x_ref, dsrc_ref, w_ref, o_ref):
    dinv = lax.rsqrt(jnp.maximum(dsrc_ref[...], 1.0))
    o_ref[...] = jnp.dot(x_ref[...] * dinv[:, None], w_ref[...],
                         preferred_element_type=jnp.float32)


def _mm_first(x, dsrc, w):
    return pl.pallas_call(
        _mm_first_body,
        grid=(NPAD // BLK,),
        in_specs=[pl.BlockSpec((BLK, D), lambda i: (i, 0)),
                  pl.BlockSpec((BLK,), lambda i: (i,)),
                  pl.BlockSpec((D, D), lambda i: (0, 0))],
        out_specs=pl.BlockSpec((BLK, D), lambda i: (i, 0)),
        out_shape=jax.ShapeDtypeStruct((NPAD, D), jnp.float32),
    )(x, dsrc, w)


def _mm_mid_body(p0_ref, p1_ref, ddst_ref, dsrc_ref, b_ref, w_ref, o_ref):
    dinvd = lax.rsqrt(jnp.maximum(ddst_ref[...], 1.0))
    dinvs = lax.rsqrt(jnp.maximum(dsrc_ref[...], 1.0))
    agg = p0_ref[...] + p1_ref[...]
    x = jnp.maximum(agg * dinvd[:, None] + b_ref[...], 0.0)
    o_ref[...] = jnp.dot(x * dinvs[:, None], w_ref[...],
                         preferred_element_type=jnp.float32)


def _mm_mid(p0, p1, ddst, dsrc, b, w):
    return pl.pallas_call(
        _mm_mid_body,
        grid=(NPAD // BLK,),
        in_specs=[pl.BlockSpec((BLK, D), lambda i: (i, 0)),
                  pl.BlockSpec((BLK, D), lambda i: (i, 0)),
                  pl.BlockSpec((BLK,), lambda i: (i,)),
                  pl.BlockSpec((BLK,), lambda i: (i,)),
                  pl.BlockSpec((1, D), lambda i: (0, 0)),
                  pl.BlockSpec((D, D), lambda i: (0, 0))],
        out_specs=pl.BlockSpec((BLK, D), lambda i: (i, 0)),
        out_shape=jax.ShapeDtypeStruct((NPAD, D), jnp.float32),
    )(p0, p1, ddst, dsrc, b, w)


def _final_body(p0_ref, p1_ref, ddst_ref, b_ref, o_ref):
    dinvd = lax.rsqrt(jnp.maximum(ddst_ref[...], 1.0))
    o_ref[...] = (p0_ref[...] + p1_ref[...]) * dinvd[:, None] + b_ref[...]


def _final(p0, p1, ddst, b):
    return pl.pallas_call(
        _final_body,
        grid=(NPAD // BLK,),
        in_specs=[pl.BlockSpec((BLK, D), lambda i: (i, 0)),
                  pl.BlockSpec((BLK, D), lambda i: (i, 0)),
                  pl.BlockSpec((BLK,), lambda i: (i,)),
                  pl.BlockSpec((1, D), lambda i: (0, 0))],
        out_specs=pl.BlockSpec((BLK, D), lambda i: (i, 0)),
        out_shape=jax.ShapeDtypeStruct((NPAD, D), jnp.float32),
    )(p0, p1, ddst, b)


def kernel(features, edge_index, W0, b0, W1, b1, W2, b2):
    pad = jnp.full((EPAD - N_EDGES,), NPAD - 1, dtype=jnp.int32)
    src_flat = jnp.concatenate([edge_index[0], pad])
    dst_flat = jnp.concatenate([edge_index[1], pad])
    srcp = src_flat.reshape(NW, CPW, CH)
    dstp = dst_flat.reshape(NW, CPW, CH)
    srcd = src_flat.reshape(16, CPS, CH)
    dstd = dst_flat.reshape(16, CPS, CH)
    featp = jnp.pad(features, ((0, NPAD - N_NODES), (0, 0)))
    ones128 = jnp.ones((CH, D), jnp.float32)
    zeros128 = jnp.zeros((CH, D), jnp.float32)

    degs = _degrees(srcd, dstd, ones128, zeros128)
    deg_src = degs[0, :, 0]
    deg_dst = degs[1, :, 0]

    h = _mm_first(featp, deg_src, W0)
    p = _aggregate(h, srcp, dstp, zeros128)
    h = _mm_mid(p[0], p[1], deg_dst, deg_src, b0[None, :], W1)
    p = _aggregate(h, srcp, dstp, zeros128)
    h = _mm_mid(p[0], p[1], deg_dst, deg_src, b1[None, :], W2)
    p = _aggregate(h, srcp, dstp, zeros128)
    out = _final(p[0], p[1], deg_dst, b2[None, :])
    return out[:N_NODES]


# speedup vs baseline: 6.4875x; 2.4635x over previous
import functools

import jax
import jax.numpy as jnp
from jax import lax
from jax.experimental import pallas as pl
from jax.experimental.pallas import tpu as pltpu
from jax.experimental.pallas import tpu_sc as plsc

N_NODES = 10000
NPAD = 10240
N_EDGES = 320000
EPAD = 327680
D = 128
NW = 32
CPW = 80
CH = 128
ROWS_PER_SUB = NPAD // 16


def _mesh():
    return plsc.VectorSubcoreMesh(core_axis_name="c", subcore_axis_name="s")


CPS = EPAD // 16 // CH


def _deg_body(src_hbm, dst_hbm, ones_hbm, zeros_hbm, out_hbm,
              idx_v, buf_v, acc):
    c = lax.axis_index("c")
    s = lax.axis_index("s")
    pltpu.sync_copy(zeros_hbm, buf_v)
    for t in range(ROWS_PER_SUB // CH):
        pltpu.sync_copy(buf_v, acc.at[pl.ds(s * ROWS_PER_SUB + t * CH, CH)])

    @pl.when(c == 0)
    def _():
        pltpu.sync_copy(src_hbm.at[s], idx_v)

    @pl.when(c == 1)
    def _():
        pltpu.sync_copy(dst_hbm.at[s], idx_v)

    pltpu.sync_copy(ones_hbm, buf_v)
    plsc.subcore_barrier()

    def body(j, carry):
        pltpu.sync_copy(buf_v, acc.at[idx_v.at[j]], add=True)
        return carry

    lax.fori_loop(0, CPS, body, 0)
    plsc.subcore_barrier()
    pltpu.sync_copy(acc.at[pl.ds(s * ROWS_PER_SUB, ROWS_PER_SUB)],
                    out_hbm.at[c, pl.ds(s * ROWS_PER_SUB, ROWS_PER_SUB)])


def _degrees(srcd, dstd, ones128, zeros128):
    k = pl.kernel(
        _deg_body,
        out_type=jax.ShapeDtypeStruct((2, NPAD, D), jnp.float32),
        mesh=_mesh(),
        scratch_types=[
            pltpu.VMEM((CPS, CH), jnp.int32),
            pltpu.VMEM((CH, D), jnp.float32),
            pltpu.VMEM_SHARED((NPAD, D), jnp.float32),
        ],
    )
    return k(srcd, dstd, ones128, zeros128)


def _agg_body(h_hbm, src_hbm, dst_hbm, zeros_hbm, out_hbm,
              src_v, dst_v, rows_v, acc, sem):
    c = lax.axis_index("c")
    s = lax.axis_index("s")
    wid = s * 2 + c
    pltpu.sync_copy(zeros_hbm, rows_v)
    for t in range(ROWS_PER_SUB // CH):
        pltpu.sync_copy(rows_v, acc.at[pl.ds(s * ROWS_PER_SUB + t * CH, CH)])
    pltpu.sync_copy(src_hbm.at[wid], src_v)
    pltpu.sync_copy(dst_hbm.at[wid], dst_v)
    plsc.subcore_barrier()

    def body(j, carry):
        pltpu.async_copy(h_hbm.at[src_v.at[j]], rows_v, sem).wait()
        pltpu.sync_copy(rows_v, acc.at[dst_v.at[j]], add=True)
        return carry

    lax.fori_loop(0, CPW, body, 0)
    plsc.subcore_barrier()
    pltpu.sync_copy(acc.at[pl.ds(s * ROWS_PER_SUB, ROWS_PER_SUB)],
                    out_hbm.at[c, pl.ds(s * ROWS_PER_SUB, ROWS_PER_SUB)])


def _aggregate(h, srcp, dstp, zeros128):
    k = pl.kernel(
        _agg_body,
        out_type=jax.ShapeDtypeStruct((2, NPAD, D), jnp.float32),
        mesh=_mesh(),
        scratch_types=[
            pltpu.VMEM((CPW, CH), jnp.int32),
            pltpu.VMEM((CPW, CH), jnp.int32),
            pltpu.VMEM((CH, D), jnp.float32),
            pltpu.VMEM_SHARED((NPAD, D), jnp.float32),
            pltpu.SemaphoreType.DMA,
        ],
    )
    return k(h, srcp, dstp, zeros128)


BLK = 1024


def _mm_first_body(x_ref, dsrc_ref, w_ref, o_ref):
    dinv = lax.rsqrt(jnp.maximum(dsrc_ref[...], 1.0))
    o_ref[...] = jnp.dot(x_ref[...] * dinv[:, None], w_ref[...],
                         preferred_element_type=jnp.float32)


def _mm_first(x, dsrc, w):
    return pl.pallas_call(
        _mm_first_body,
        grid=(NPAD // BLK,),
        in_specs=[pl.BlockSpec((BLK, D), lambda i: (i, 0)),
                  pl.BlockSpec((BLK,), lambda i: (i,)),
                  pl.BlockSpec((D, D), lambda i: (0, 0))],
        out_specs=pl.BlockSpec((BLK, D), lambda i: (i, 0)),
        out_shape=jax.ShapeDtypeStruct((NPAD, D), jnp.float32),
    )(x, dsrc, w)


def _mm_mid_body(p0_ref, p1_ref, ddst_ref, dsrc_ref, b_ref, w_ref, o_ref):
    dinvd = lax.rsqrt(jnp.maximum(ddst_ref[...], 1.0))
    dinvs = lax.rsqrt(jnp.maximum(dsrc_ref[...], 1.0))
    agg = p0_ref[...] + p1_ref[...]
    x = jnp.maximum(agg * dinvd[:, None] + b_ref[...], 0.0)
    o_ref[...] = jnp.dot(x * dinvs[:, None], w_ref[...],
                         preferred_element_type=jnp.float32)


def _mm_mid(p0, p1, ddst, dsrc, b, w):
    return pl.pallas_call(
        _mm_mid_body,
        grid=(NPAD // BLK,),
        in_specs=[pl.BlockSpec((BLK, D), lambda i: (i, 0)),
                  pl.BlockSpec((BLK, D), lambda i: (i, 0)),
                  pl.BlockSpec((BLK,), lambda i: (i,)),
                  pl.BlockSpec((BLK,), lambda i: (i,)),
                  pl.BlockSpec((1, D), lambda i: (0, 0)),
                  pl.BlockSpec((D, D), lambda i: (0, 0))],
        out_specs=pl.BlockSpec((BLK, D), lambda i: (i, 0)),
        out_shape=jax.ShapeDtypeStruct((NPAD, D), jnp.float32),
    )(p0, p1, ddst, dsrc, b, w)


def _final_body(p0_ref, p1_ref, ddst_ref, b_ref, o_ref):
    dinvd = lax.rsqrt(jnp.maximum(ddst_ref[...], 1.0))
    o_ref[...] = (p0_ref[...] + p1_ref[...]) * dinvd[:, None] + b_ref[...]


def _final(p0, p1, ddst, b):
    return pl.pallas_call(
        _final_body,
        grid=(NPAD // BLK,),
        in_specs=[pl.BlockSpec((BLK, D), lambda i: (i, 0)),
                  pl.BlockSpec((BLK, D), lambda i: (i, 0)),
                  pl.BlockSpec((BLK,), lambda i: (i,)),
                  pl.BlockSpec((1, D), lambda i: (0, 0))],
        out_specs=pl.BlockSpec((BLK, D), lambda i: (i, 0)),
        out_shape=jax.ShapeDtypeStruct((NPAD, D), jnp.float32),
    )(p0, p1, ddst, b)


def kernel(features, edge_index, W0, b0, W1, b1, W2, b2):
    pad = (N_NODES + jnp.arange(EPAD - N_EDGES, dtype=jnp.int32)
           % (NPAD - N_NODES))
    src_flat = jnp.concatenate([edge_index[0], pad])
    dst_flat = jnp.concatenate([edge_index[1], pad])
    srcp = src_flat.reshape(NW, CPW, CH)
    dstp = dst_flat.reshape(NW, CPW, CH)
    srcd = src_flat.reshape(16, CPS, CH)
    dstd = dst_flat.reshape(16, CPS, CH)
    featp = jnp.pad(features, ((0, NPAD - N_NODES), (0, 0)))
    ones128 = jnp.ones((CH, D), jnp.float32)
    zeros128 = jnp.zeros((CH, D), jnp.float32)

    degs = _degrees(srcd, dstd, ones128, zeros128)
    deg_src = degs[0, :, 0]
    deg_dst = degs[1, :, 0]

    h = _mm_first(featp, deg_src, W0)
    p = _aggregate(h, srcp, dstp, zeros128)
    h = _mm_mid(p[0], p[1], deg_dst, deg_src, b0[None, :], W1)
    p = _aggregate(h, srcp, dstp, zeros128)
    h = _mm_mid(p[0], p[1], deg_dst, deg_src, b1[None, :], W2)
    p = _aggregate(h, srcp, dstp, zeros128)
    out = _final(p[0], p[1], deg_dst, b2[None, :])
    return out[:N_NODES]


# speedup vs baseline: 7.4376x; 1.1465x over previous
import functools

import jax
import jax.numpy as jnp
from jax import lax
from jax.experimental import pallas as pl
from jax.experimental.pallas import tpu as pltpu
from jax.experimental.pallas import tpu_sc as plsc

N_NODES = 10000
NPAD = 10240
N_EDGES = 320000
D = 128
NW = 32
CPW = 80
CH = 128
WND = 8
EPAD = NW * CPW * CH
ROWS_PER_SUB = NPAD // 16


def _mesh():
    return plsc.VectorSubcoreMesh(core_axis_name="c", subcore_axis_name="s")


CPS = EPAD // 16 // CH


def _deg_body(src_hbm, dst_hbm, ones_hbm, zeros_hbm, out_hbm,
              idx_v, buf_v, acc):
    c = lax.axis_index("c")
    s = lax.axis_index("s")
    pltpu.sync_copy(zeros_hbm, buf_v)
    for t in range(ROWS_PER_SUB // CH):
        pltpu.sync_copy(buf_v, acc.at[pl.ds(s * ROWS_PER_SUB + t * CH, CH)])

    @pl.when(c == 0)
    def _():
        pltpu.sync_copy(src_hbm.at[s], idx_v)

    @pl.when(c == 1)
    def _():
        pltpu.sync_copy(dst_hbm.at[s], idx_v)

    pltpu.sync_copy(ones_hbm, buf_v)
    plsc.subcore_barrier()

    def body(j, carry):
        pltpu.sync_copy(buf_v, acc.at[idx_v.at[j]], add=True)
        return carry

    lax.fori_loop(0, CPS, body, 0)
    plsc.subcore_barrier()
    pltpu.sync_copy(acc.at[pl.ds(s * ROWS_PER_SUB, ROWS_PER_SUB)],
                    out_hbm.at[c, pl.ds(s * ROWS_PER_SUB, ROWS_PER_SUB)])


def _degrees(srcd, dstd, ones128, zeros128):
    k = pl.kernel(
        _deg_body,
        out_type=jax.ShapeDtypeStruct((2, NPAD, D), jnp.float32),
        mesh=_mesh(),
        scratch_types=[
            pltpu.VMEM((CPS, CH), jnp.int32),
            pltpu.VMEM((CH, D), jnp.float32),
            pltpu.VMEM_SHARED((NPAD, D), jnp.float32),
        ],
    )
    return k(srcd, dstd, ones128, zeros128)


def _agg_body(h_hbm, src_hbm, dst_hbm, zeros_hbm, out_hbm,
              srcw, dstw, rows_a, rows_b, acc, sem_a, sem_b):
    c = lax.axis_index("c")
    s = lax.axis_index("s")
    wid = s * 2 + c
    pltpu.sync_copy(zeros_hbm, rows_a)
    for t in range(ROWS_PER_SUB // CH):
        pltpu.sync_copy(rows_a, acc.at[pl.ds(s * ROWS_PER_SUB + t * CH, CH)])
    plsc.subcore_barrier()

    def window(w, carry):
        pltpu.sync_copy(src_hbm.at[wid, pl.ds(w * WND, WND)], srcw)
        pltpu.sync_copy(dst_hbm.at[wid, pl.ds(w * WND, WND)], dstw)
        bufs = (rows_a, rows_b)
        sems = (sem_a, sem_b)
        pltpu.async_copy(h_hbm.at[srcw.at[0]], rows_a, sem_a)
        for k in range(WND):
            b, sm = bufs[k % 2], sems[k % 2]
            pltpu.make_async_copy(h_hbm.at[srcw.at[k]], b, sm).wait()
            if k + 1 < WND:
                pltpu.async_copy(h_hbm.at[srcw.at[k + 1]],
                                 bufs[(k + 1) % 2], sems[(k + 1) % 2])
            pltpu.sync_copy(b, acc.at[dstw.at[k]], add=True)
        return carry

    lax.fori_loop(0, CPW // WND, window, 0)
    plsc.subcore_barrier()
    pltpu.sync_copy(acc.at[pl.ds(s * ROWS_PER_SUB, ROWS_PER_SUB)],
                    out_hbm.at[c, pl.ds(s * ROWS_PER_SUB, ROWS_PER_SUB)])


def _aggregate(h, srcp, dstp, zeros128):
    k = pl.kernel(
        _agg_body,
        out_type=jax.ShapeDtypeStruct((2, NPAD, D), jnp.float32),
        mesh=_mesh(),
        scratch_types=[
            pltpu.VMEM((WND, CH), jnp.int32),
            pltpu.VMEM((WND, CH), jnp.int32),
            pltpu.VMEM((CH, D), jnp.float32),
            pltpu.VMEM((CH, D), jnp.float32),
            pltpu.VMEM_SHARED((NPAD, D), jnp.float32),
            pltpu.SemaphoreType.DMA,
            pltpu.SemaphoreType.DMA,
        ],
    )
    return k(h, srcp, dstp, zeros128)


BLK = 1024


def _mm_first_body(x_ref, dsrc_ref, w_ref, o_ref):
    dinv = lax.rsqrt(jnp.maximum(dsrc_ref[...], 1.0))
    o_ref[...] = jnp.dot(x_ref[...] * dinv[:, None], w_ref[...],
                         preferred_element_type=jnp.float32)


def _mm_first(x, dsrc, w):
    return pl.pallas_call(
        _mm_first_body,
        grid=(NPAD // BLK,),
        in_specs=[pl.BlockSpec((BLK, D), lambda i: (i, 0)),
                  pl.BlockSpec((BLK,), lambda i: (i,)),
                  pl.BlockSpec((D, D), lambda i: (0, 0))],
        out_specs=pl.BlockSpec((BLK, D), lambda i: (i, 0)),
        out_shape=jax.ShapeDtypeStruct((NPAD, D), jnp.float32),
    )(x, dsrc, w)


def _mm_mid_body(p0_ref, p1_ref, ddst_ref, dsrc_ref, b_ref, w_ref, o_ref):
    dinvd = lax.rsqrt(jnp.maximum(ddst_ref[...], 1.0))
    dinvs = lax.rsqrt(jnp.maximum(dsrc_ref[...], 1.0))
    agg = p0_ref[...] + p1_ref[...]
    x = jnp.maximum(agg * dinvd[:, None] + b_ref[...], 0.0)
    o_ref[...] = jnp.dot(x * dinvs[:, None], w_ref[...],
                         preferred_element_type=jnp.float32)


def _mm_mid(p0, p1, ddst, dsrc, b, w):
    return pl.pallas_call(
        _mm_mid_body,
        grid=(NPAD // BLK,),
        in_specs=[pl.BlockSpec((BLK, D), lambda i: (i, 0)),
                  pl.BlockSpec((BLK, D), lambda i: (i, 0)),
                  pl.BlockSpec((BLK,), lambda i: (i,)),
                  pl.BlockSpec((BLK,), lambda i: (i,)),
                  pl.BlockSpec((1, D), lambda i: (0, 0)),
                  pl.BlockSpec((D, D), lambda i: (0, 0))],
        out_specs=pl.BlockSpec((BLK, D), lambda i: (i, 0)),
        out_shape=jax.ShapeDtypeStruct((NPAD, D), jnp.float32),
    )(p0, p1, ddst, dsrc, b, w)


def _final_body(p0_ref, p1_ref, ddst_ref, b_ref, o_ref):
    dinvd = lax.rsqrt(jnp.maximum(ddst_ref[...], 1.0))
    o_ref[...] = (p0_ref[...] + p1_ref[...]) * dinvd[:, None] + b_ref[...]


def _final(p0, p1, ddst, b):
    return pl.pallas_call(
        _final_body,
        grid=(NPAD // BLK,),
        in_specs=[pl.BlockSpec((BLK, D), lambda i: (i, 0)),
                  pl.BlockSpec((BLK, D), lambda i: (i, 0)),
                  pl.BlockSpec((BLK,), lambda i: (i,)),
                  pl.BlockSpec((1, D), lambda i: (0, 0))],
        out_specs=pl.BlockSpec((BLK, D), lambda i: (i, 0)),
        out_shape=jax.ShapeDtypeStruct((NPAD, D), jnp.float32),
    )(p0, p1, ddst, b)


def kernel(features, edge_index, W0, b0, W1, b1, W2, b2):
    pad = (N_NODES + jnp.arange(EPAD - N_EDGES, dtype=jnp.int32)
           % (NPAD - N_NODES))
    src_flat = jnp.concatenate([edge_index[0], pad])
    dst_flat = jnp.concatenate([edge_index[1], pad])
    srcp = src_flat.reshape(NW, CPW, CH)
    dstp = dst_flat.reshape(NW, CPW, CH)
    srcd = src_flat.reshape(16, CPS, CH)
    dstd = dst_flat.reshape(16, CPS, CH)
    featp = jnp.pad(features, ((0, NPAD - N_NODES), (0, 0)))
    ones128 = jnp.ones((CH, D), jnp.float32)
    zeros128 = jnp.zeros((CH, D), jnp.float32)

    degs = _degrees(srcd, dstd, ones128, zeros128)
    deg_src = degs[0, :, 0]
    deg_dst = degs[1, :, 0]

    h = _mm_first(featp, deg_src, W0)
    p = _aggregate(h, srcp, dstp, zeros128)
    h = _mm_mid(p[0], p[1], deg_dst, deg_src, b0[None, :], W1)
    p = _aggregate(h, srcp, dstp, zeros128)
    h = _mm_mid(p[0], p[1], deg_dst, deg_src, b1[None, :], W2)
    p = _aggregate(h, srcp, dstp, zeros128)
    out = _final(p[0], p[1], deg_dst, b2[None, :])
    return out[:N_NODES]


# speedup vs baseline: 7.4478x; 1.0014x over previous
import functools

import jax
import jax.numpy as jnp
from jax import lax
from jax.experimental import pallas as pl
from jax.experimental.pallas import tpu as pltpu
from jax.experimental.pallas import tpu_sc as plsc

N_NODES = 10000
NPAD = 10240
N_EDGES = 320000
D = 128
NW = 32
CPW = 80
CH = 128
WND = 8
EPAD = NW * CPW * CH
ROWS_PER_SUB = NPAD // 16


def _mesh():
    return plsc.VectorSubcoreMesh(core_axis_name="c", subcore_axis_name="s")


CPS = EPAD // 16 // CH


def _deg_body(src_hbm, dst_hbm, ones_hbm, zeros_hbm, out_hbm,
              idx_v, buf_v, acc, s0, s1, s2, s3):
    sems = (s0, s1, s2, s3)
    c = lax.axis_index("c")
    s = lax.axis_index("s")
    pltpu.sync_copy(zeros_hbm, buf_v)
    for t in range(ROWS_PER_SUB // CH):
        pltpu.sync_copy(buf_v, acc.at[pl.ds(s * ROWS_PER_SUB + t * CH, CH)])

    @pl.when(c == 0)
    def _():
        pltpu.sync_copy(src_hbm.at[s], idx_v)

    @pl.when(c == 1)
    def _():
        pltpu.sync_copy(dst_hbm.at[s], idx_v)

    pltpu.sync_copy(ones_hbm, buf_v)
    plsc.subcore_barrier()

    def body(i, carry):
        j0 = 4 * i
        for u in range(4):
            pltpu.async_copy(buf_v, acc.at[idx_v.at[j0 + u]], sems[u],
                             add=True)
        for u in range(4):
            pltpu.make_async_copy(buf_v, acc.at[idx_v.at[j0 + u]],
                                  sems[u]).wait()
        return carry

    lax.fori_loop(0, CPS // 4, body, 0)
    plsc.subcore_barrier()
    pltpu.sync_copy(acc.at[pl.ds(s * ROWS_PER_SUB, ROWS_PER_SUB)],
                    out_hbm.at[c, pl.ds(s * ROWS_PER_SUB, ROWS_PER_SUB)])


def _degrees(srcd, dstd, ones128, zeros128):
    k = pl.kernel(
        _deg_body,
        out_type=jax.ShapeDtypeStruct((2, NPAD, D), jnp.float32),
        mesh=_mesh(),
        scratch_types=[
            pltpu.VMEM((CPS, CH), jnp.int32),
            pltpu.VMEM((CH, D), jnp.float32),
            pltpu.VMEM_SHARED((NPAD, D), jnp.float32),
            pltpu.SemaphoreType.DMA,
            pltpu.SemaphoreType.DMA,
            pltpu.SemaphoreType.DMA,
            pltpu.SemaphoreType.DMA,
        ],
    )
    return k(srcd, dstd, ones128, zeros128)


def _agg_body(h_hbm, src_hbm, dst_hbm, zeros_hbm, out_hbm,
              srcw, dstw, rows_a, rows_b, acc, sem_a, sem_b, sem_c, sem_d):
    c = lax.axis_index("c")
    s = lax.axis_index("s")
    wid = s * 2 + c
    pltpu.sync_copy(zeros_hbm, rows_a)
    for t in range(ROWS_PER_SUB // CH):
        pltpu.sync_copy(rows_a, acc.at[pl.ds(s * ROWS_PER_SUB + t * CH, CH)])
    plsc.subcore_barrier()

    def window(w, carry):
        pltpu.sync_copy(src_hbm.at[wid, pl.ds(w * WND, WND)], srcw)
        pltpu.sync_copy(dst_hbm.at[wid, pl.ds(w * WND, WND)], dstw)
        bufs = (rows_a, rows_b)
        gsem = (sem_a, sem_b)
        ssem = (sem_c, sem_d)
        pltpu.async_copy(h_hbm.at[srcw.at[0]], rows_a, sem_a)
        for k in range(WND):
            p = k % 2
            q = (k + 1) % 2
            pltpu.make_async_copy(h_hbm.at[srcw.at[k]], bufs[p],
                                  gsem[p]).wait()
            pltpu.async_copy(bufs[p], acc.at[dstw.at[k]], ssem[p], add=True)
            if k > 0:
                pltpu.make_async_copy(bufs[q], acc.at[dstw.at[k - 1]],
                                      ssem[q]).wait()
            if k + 1 < WND:
                pltpu.async_copy(h_hbm.at[srcw.at[k + 1]], bufs[q], gsem[q])
        pltpu.make_async_copy(bufs[(WND - 1) % 2], acc.at[dstw.at[WND - 1]],
                              ssem[(WND - 1) % 2]).wait()
        return carry

    lax.fori_loop(0, CPW // WND, window, 0)
    plsc.subcore_barrier()
    pltpu.sync_copy(acc.at[pl.ds(s * ROWS_PER_SUB, ROWS_PER_SUB)],
                    out_hbm.at[c, pl.ds(s * ROWS_PER_SUB, ROWS_PER_SUB)])


def _aggregate(h, srcp, dstp, zeros128):
    k = pl.kernel(
        _agg_body,
        out_type=jax.ShapeDtypeStruct((2, NPAD, D), jnp.float32),
        mesh=_mesh(),
        scratch_types=[
            pltpu.VMEM((WND, CH), jnp.int32),
            pltpu.VMEM((WND, CH), jnp.int32),
            pltpu.VMEM((CH, D), jnp.float32),
            pltpu.VMEM((CH, D), jnp.float32),
            pltpu.VMEM_SHARED((NPAD, D), jnp.float32),
            pltpu.SemaphoreType.DMA,
            pltpu.SemaphoreType.DMA,
            pltpu.SemaphoreType.DMA,
            pltpu.SemaphoreType.DMA,
        ],
    )
    return k(h, srcp, dstp, zeros128)


BLK = 1024


def _mm_first_body(x_ref, dsrc_ref, w_ref, o_ref):
    dinv = lax.rsqrt(jnp.maximum(dsrc_ref[...], 1.0))
    o_ref[...] = jnp.dot(x_ref[...] * dinv[:, None], w_ref[...],
                         preferred_element_type=jnp.float32)


def _mm_first(x, dsrc, w):
    return pl.pallas_call(
        _mm_first_body,
        grid=(NPAD // BLK,),
        in_specs=[pl.BlockSpec((BLK, D), lambda i: (i, 0)),
                  pl.BlockSpec((BLK,), lambda i: (i,)),
                  pl.BlockSpec((D, D), lambda i: (0, 0))],
        out_specs=pl.BlockSpec((BLK, D), lambda i: (i, 0)),
        out_shape=jax.ShapeDtypeStruct((NPAD, D), jnp.float32),
    )(x, dsrc, w)


def _mm_mid_body(p0_ref, p1_ref, ddst_ref, dsrc_ref, b_ref, w_ref, o_ref):
    dinvd = lax.rsqrt(jnp.maximum(ddst_ref[...], 1.0))
    dinvs = lax.rsqrt(jnp.maximum(dsrc_ref[...], 1.0))
    agg = p0_ref[...] + p1_ref[...]
    x = jnp.maximum(agg * dinvd[:, None] + b_ref[...], 0.0)
    o_ref[...] = jnp.dot(x * dinvs[:, None], w_ref[...],
                         preferred_element_type=jnp.float32)


def _mm_mid(p0, p1, ddst, dsrc, b, w):
    return pl.pallas_call(
        _mm_mid_body,
        grid=(NPAD // BLK,),
        in_specs=[pl.BlockSpec((BLK, D), lambda i: (i, 0)),
                  pl.BlockSpec((BLK, D), lambda i: (i, 0)),
                  pl.BlockSpec((BLK,), lambda i: (i,)),
                  pl.BlockSpec((BLK,), lambda i: (i,)),
                  pl.BlockSpec((1, D), lambda i: (0, 0)),
                  pl.BlockSpec((D, D), lambda i: (0, 0))],
        out_specs=pl.BlockSpec((BLK, D), lambda i: (i, 0)),
        out_shape=jax.ShapeDtypeStruct((NPAD, D), jnp.float32),
    )(p0, p1, ddst, dsrc, b, w)


def _final_body(p0_ref, p1_ref, ddst_ref, b_ref, o_ref):
    dinvd = lax.rsqrt(jnp.maximum(ddst_ref[...], 1.0))
    o_ref[...] = (p0_ref[...] + p1_ref[...]) * dinvd[:, None] + b_ref[...]


def _final(p0, p1, ddst, b):
    return pl.pallas_call(
        _final_body,
        grid=(NPAD // BLK,),
        in_specs=[pl.BlockSpec((BLK, D), lambda i: (i, 0)),
                  pl.BlockSpec((BLK, D), lambda i: (i, 0)),
                  pl.BlockSpec((BLK,), lambda i: (i,)),
                  pl.BlockSpec((1, D), lambda i: (0, 0))],
        out_specs=pl.BlockSpec((BLK, D), lambda i: (i, 0)),
        out_shape=jax.ShapeDtypeStruct((NPAD, D), jnp.float32),
    )(p0, p1, ddst, b)


def kernel(features, edge_index, W0, b0, W1, b1, W2, b2):
    pad = (N_NODES + jnp.arange(EPAD - N_EDGES, dtype=jnp.int32)
           % (NPAD - N_NODES))
    src_flat = jnp.concatenate([edge_index[0], pad])
    dst_flat = jnp.concatenate([edge_index[1], pad])
    srcp = src_flat.reshape(NW, CPW, CH)
    dstp = dst_flat.reshape(NW, CPW, CH)
    srcd = src_flat.reshape(16, CPS, CH)
    dstd = dst_flat.reshape(16, CPS, CH)
    featp = jnp.pad(features, ((0, NPAD - N_NODES), (0, 0)))
    ones128 = jnp.ones((CH, D), jnp.float32)
    zeros128 = jnp.zeros((CH, D), jnp.float32)

    degs = _degrees(srcd, dstd, ones128, zeros128)
    deg_src = degs[0, :, 0]
    deg_dst = degs[1, :, 0]

    h = _mm_first(featp, deg_src, W0)
    p = _aggregate(h, srcp, dstp, zeros128)
    h = _mm_mid(p[0], p[1], deg_dst, deg_src, b0[None, :], W1)
    p = _aggregate(h, srcp, dstp, zeros128)
    h = _mm_mid(p[0], p[1], deg_dst, deg_src, b1[None, :], W2)
    p = _aggregate(h, srcp, dstp, zeros128)
    out = _final(p[0], p[1], deg_dst, b2[None, :])
    return out[:N_NODES]
